# baseline (speedup 1.0000x reference)
import functools

import numpy as np

import jax
import jax.numpy as jnp
from jax import lax
from jax.experimental import pallas as pl
from jax.experimental.pallas import tpu as pltpu
from jax.experimental.pallas import tpu_sc as plsc

B, N, K, C, SX, SO = 4, 400, 32, 6, 12, 12
F = 16
H = 48
R = 4
M = R * K
RB = 25
NB = RB * R
MB = NB * K
GRID = (B * N) // NB
OUTW = C * SO

NC, NS = 2, 16
NW = NC * NS
PER_W = (B * N * K) // NW
CH, CW = 16, 100


def _gather_body(table_hbm, idx_hbm, out_hbm, idx_v, rows_v, sem):
    wid = lax.axis_index("s") * NC + lax.axis_index("c")
    batch = wid // (NW // B)
    pltpu.sync_copy(idx_hbm.at[wid], idx_v)
    copies = [
        pltpu.async_copy(
            table_hbm.at[batch].at[idx_v.at[j]],
            rows_v.at[pl.ds(j * CW, CW)],
            sem,
        )
        for j in range(CH)
    ]
    for cp in copies:
        cp.wait()
    pltpu.sync_copy(rows_v, out_hbm.at[pl.ds(wid * PER_W, PER_W)])


def _sc_gather(table, idx3):
    mesh = plsc.VectorSubcoreMesh(core_axis_name="c", subcore_axis_name="s")
    run = functools.partial(
        pl.kernel,
        out_type=jax.ShapeDtypeStruct((B * N * K, F), jnp.float32),
        mesh=mesh,
        scratch_types=[
            pltpu.VMEM((CH, CW), jnp.int32),
            pltpu.VMEM((PER_W, F), jnp.float32),
            pltpu.SemaphoreType.DMA,
        ],
        compiler_params=pltpu.CompilerParams(use_tc_tiling_on_sc=False),
    )(_gather_body)
    return run(table, idx3)


_mm = np.arange(MB)
_SEL_T = (np.arange(NB)[:, None] == _mm[None, :] // K).astype(np.float32)
_m1 = np.arange(M)
_blk = (_m1[:, None] // K) == (_m1[None, :] // K)
_W1 = (_blk & (_m1[:, None] != _m1[None, :])).astype(np.float32)


def _tc_body(x_ref, g_ref, ww_ref, wa1_ref, wa2_ref, bw_ref, ba1_ref,
             ba2_ref, selt_ref, w1_ref, out_ref, acc_ref):
    i = pl.program_id(0)

    @pl.when(i == 0)
    def _init():
        acc_ref[...] = jnp.zeros_like(acc_ref)

    def leaky(v):
        return jnp.maximum(v, 0.5 * v)

    def dot(a, b):
        return jnp.dot(a, b, preferred_element_type=jnp.float32)

    def dot_nt(a, b):
        return lax.dot_general(a, b, (((1,), (1,)), ((), ())),
                               preferred_element_type=jnp.float32)

    def dot_tn(a, b):
        return lax.dot_general(a, b, (((0,), (0,)), ((), ())),
                               preferred_element_type=jnp.float32)

    x = x_ref[0]
    g = g_ref[0]
    sel_t = selt_ref[...]

    ww = ww_ref[...]
    wa1 = wa1_ref[...]
    wa1x = wa1[:, :SO]
    wa1y = wa1[:, SO:]
    wa2 = wa2_ref[...]
    bw = bw_ref[...]
    ba1 = ba1_ref[...]
    ba2 = ba2_ref[...]

    wht = leaky(dot_nt(ww, x[:, :SX]) + bw)
    whgt = leaky(dot_nt(ww, g[:, :SX]) + bw)

    pxt = dot(wa1x, wht) + ba1
    pxrep = dot(pxt, sel_t)
    pyt = dot(wa1y, whgt)
    ht = leaky(pxrep + pyt)
    att = leaky(dot(wa2, ht) + ba2)

    mx = jnp.max(att, axis=0, keepdims=True)
    e = jnp.exp(att - mx)
    amt = e / jnp.sum(e, axis=0, keepdims=True)

    outs = [dot_nt(sel_t, whgt * amt[c:c + 1]) for c in range(C)]
    out_ref[0] = jnp.concatenate(outs, axis=1)

    ns = dot_nt(sel_t, whgt)
    sq = jnp.sum(whgt * whgt, axis=0, keepdims=True)
    dist_t = -2.0 * jnp.sum(ns * ns) + 2.0 * K * jnp.sum(sq)

    w1 = w1_ref[...]
    one11 = jnp.ones((1, 1), dtype=jnp.float32)
    loss_t = 0.0
    for b in range(RB):
        amb = amt[:, b * M:(b + 1) * M]
        whb = whgt[:, b * M:(b + 1) * M]
        sqb = sq[:, b * M:(b + 1) * M]
        prob = dot_tn(amb, amb)
        gram = dot_tn(whb, whb)
        sqcol = dot_tn(sqb, one11)
        dist = -2.0 * gram + sqcol + sqb
        sign = jnp.where(dist <= 0.2, 1.0, -1.0)
        lp = jnp.log(jnp.clip(prob, 0.0001, 1.0 - 0.0001)) * w1
        loss_t += jnp.sum(sign * lp)

    loss_sum = -loss_t
    wh_sum = jnp.sum(wht)

    acc_lane = lax.broadcasted_iota(jnp.int32, (1, 128), 1)
    vec = jnp.where(acc_lane == 0, loss_sum,
                    jnp.where(acc_lane == 1, dist_t,
                              jnp.where(acc_lane == 2, wh_sum, 0.0)))
    acc_ref[...] += vec


_CONST0 = lambda i: (0, 0)
_TC_GRID_SPEC = dict(
    grid=(GRID,),
    in_specs=[
        pl.BlockSpec((1, NB, F), lambda i: (i, 0, 0)),
        pl.BlockSpec((1, MB, F), lambda i: (i, 0, 0)),
        pl.BlockSpec((SO, SX), _CONST0),
        pl.BlockSpec((H, 2 * SO), _CONST0),
        pl.BlockSpec((C, H), _CONST0),
        pl.BlockSpec((SO, 1), _CONST0),
        pl.BlockSpec((H, 1), _CONST0),
        pl.BlockSpec((C, 1), _CONST0),
        pl.BlockSpec((NB, MB), _CONST0),
        pl.BlockSpec((M, M), _CONST0),
    ],
    out_specs=[
        pl.BlockSpec((1, NB, OUTW), lambda i: (i, 0, 0)),
        pl.BlockSpec((1, 128), _CONST0),
    ],
    out_shape=[
        jax.ShapeDtypeStruct((GRID, NB, OUTW), jnp.float32),
        jax.ShapeDtypeStruct((1, 128), jnp.float32),
    ],
)


def _tc_compute(x3, g3, Ww, bw, Wa1, ba1, Wa2, ba2):
    return pl.pallas_call(_tc_body, **_TC_GRID_SPEC)(
        x3, g3, Ww, Wa1, Wa2, bw.reshape(SO, 1), ba1.reshape(H, 1),
        ba2.reshape(C, 1), jnp.asarray(_SEL_T), jnp.asarray(_W1))


def kernel(fushed_features, input_data, Ww, bw, Wa1, ba1, Wa2, ba2, adj_idx):
    xpad = jnp.pad(input_data.reshape(B * N, SX), ((0, 0), (0, F - SX)))
    idx3 = adj_idx.astype(jnp.int32).reshape(NW, CH, CW)

    g = _sc_gather(xpad.reshape(B, N, F), idx3)

    x3 = xpad.reshape(GRID, NB, F)
    g3 = g.reshape(GRID, MB, F)

    out3, acc = _tc_compute(x3, g3, Ww, bw, Wa1, ba1, Wa2, ba2)

    output_data = out3.reshape(B, N, C, SO)
    cluster_loss = acc[0, 0] / (B * N)
    dist_mean = acc[0, 1] / (B * N * K * K)
    wh_mean = acc[0, 2] / (B * N * SO)
    return output_data, cluster_loss, dist_mean, wh_mean


# speedup vs baseline: 1.0166x; 1.0166x over previous
import functools

import numpy as np

import jax
import jax.numpy as jnp
from jax import lax
from jax.experimental import pallas as pl
from jax.experimental.pallas import tpu as pltpu
from jax.experimental.pallas import tpu_sc as plsc

B, N, K, C, SX, SO = 4, 400, 32, 6, 12, 12
F = 16
H = 48
R = 4
M = R * K
RB = 25
NB = RB * R
MB = NB * K
GRID = (B * N) // NB
OUTW = C * SO

NC, NS = 2, 16
NW = NC * NS
PER_W = (B * N * K) // NW
CH, CW = 16, 100


def _gather_body(table_hbm, idx_hbm, out_hbm, idx_v, rows_v, sem):
    wid = lax.axis_index("s") * NC + lax.axis_index("c")
    pltpu.sync_copy(idx_hbm.at[wid], idx_v)
    copies = [
        pltpu.async_copy(
            table_hbm.at[idx_v.at[j]],
            rows_v.at[pl.ds(j * CW, CW)],
            sem,
        )
        for j in range(CH)
    ]
    for cp in copies:
        cp.wait()
    pltpu.sync_copy(rows_v, out_hbm.at[pl.ds(wid * PER_W, PER_W)])


def _sc_gather(table, idx3):
    mesh = plsc.VectorSubcoreMesh(core_axis_name="c", subcore_axis_name="s")
    run = functools.partial(
        pl.kernel,
        out_type=jax.ShapeDtypeStruct((B * N * K, F), jnp.float32),
        mesh=mesh,
        scratch_types=[
            pltpu.VMEM((CH, CW), jnp.int32),
            pltpu.VMEM((PER_W, F), jnp.float32),
            pltpu.SemaphoreType.DMA,
        ],
        compiler_params=pltpu.CompilerParams(use_tc_tiling_on_sc=False),
    )(_gather_body)
    return run(table, idx3)


_mm = np.arange(MB)
_SEL_T = (np.arange(NB)[:, None] == _mm[None, :] // K).astype(np.float32)
_m1 = np.arange(M)
_blk = (_m1[:, None] // K) == (_m1[None, :] // K)
_W1 = (_blk & (_m1[:, None] != _m1[None, :])).astype(np.float32)


def _tc_body(x_ref, g_ref, ww_ref, wa1_ref, wa2_ref, bw_ref, ba1_ref,
             ba2_ref, selt_ref, w1_ref, out_ref, acc_ref):
    i = pl.program_id(0)

    @pl.when(i == 0)
    def _init():
        acc_ref[...] = jnp.zeros_like(acc_ref)

    def leaky(v):
        return jnp.maximum(v, 0.5 * v)

    def dot(a, b):
        return jnp.dot(a, b, preferred_element_type=jnp.float32)

    def dot_nt(a, b):
        return lax.dot_general(a, b, (((1,), (1,)), ((), ())),
                               preferred_element_type=jnp.float32)

    def dot_tn(a, b):
        return lax.dot_general(a, b, (((0,), (0,)), ((), ())),
                               preferred_element_type=jnp.float32)

    x = x_ref[0]
    g = g_ref[0]
    sel_t = selt_ref[...]

    ww = ww_ref[...]
    wa1 = wa1_ref[...]
    wa1x = wa1[:, :SO]
    wa1y = wa1[:, SO:]
    wa2 = wa2_ref[...]
    bw = bw_ref[...]
    ba1 = ba1_ref[...]
    ba2 = ba2_ref[...]

    wht = leaky(dot_nt(ww, x[:, :SX]) + bw)
    whgt = leaky(dot_nt(ww, g[:, :SX]) + bw)

    pxt = dot(wa1x, wht) + ba1
    pxrep = dot(pxt, sel_t)
    pyt = dot(wa1y, whgt)
    ht = leaky(pxrep + pyt)
    att = leaky(dot(wa2, ht) + ba2)

    mx = jnp.max(att, axis=0, keepdims=True)
    e = jnp.exp(att - mx)
    amt = e / jnp.sum(e, axis=0, keepdims=True)

    outs = [dot_nt(sel_t, whgt * amt[c:c + 1]) for c in range(C)]
    out_ref[0] = jnp.concatenate(outs, axis=1)

    ns = dot_nt(sel_t, whgt)
    sq = jnp.sum(whgt * whgt, axis=0, keepdims=True)
    dist_t = -2.0 * jnp.sum(ns * ns) + 2.0 * K * jnp.sum(sq)

    w1 = w1_ref[...]
    one11 = jnp.ones((1, 1), dtype=jnp.float32)
    loss_t = 0.0
    for b in range(RB):
        amb = amt[:, b * M:(b + 1) * M]
        whb = whgt[:, b * M:(b + 1) * M]
        sqb = sq[:, b * M:(b + 1) * M]
        prob = dot_tn(amb, amb)
        gram = dot_tn(whb, whb)
        sqcol = dot_tn(sqb, one11)
        dist = -2.0 * gram + sqcol + sqb
        sign = jnp.where(dist <= 0.2, 1.0, -1.0)
        lp = jnp.log(jnp.clip(prob, 0.0001, 1.0 - 0.0001)) * w1
        loss_t += jnp.sum(sign * lp)

    loss_sum = -loss_t
    wh_sum = jnp.sum(wht)

    acc_lane = lax.broadcasted_iota(jnp.int32, (1, 128), 1)
    vec = jnp.where(acc_lane == 0, loss_sum,
                    jnp.where(acc_lane == 1, dist_t,
                              jnp.where(acc_lane == 2, wh_sum, 0.0)))
    acc_ref[...] += vec


_CONST0 = lambda i: (0, 0)
_TC_GRID_SPEC = dict(
    grid=(GRID,),
    in_specs=[
        pl.BlockSpec((1, NB, F), lambda i: (i, 0, 0)),
        pl.BlockSpec((1, MB, F), lambda i: (i, 0, 0)),
        pl.BlockSpec((SO, SX), _CONST0),
        pl.BlockSpec((H, 2 * SO), _CONST0),
        pl.BlockSpec((C, H), _CONST0),
        pl.BlockSpec((SO, 1), _CONST0),
        pl.BlockSpec((H, 1), _CONST0),
        pl.BlockSpec((C, 1), _CONST0),
        pl.BlockSpec((NB, MB), _CONST0),
        pl.BlockSpec((M, M), _CONST0),
    ],
    out_specs=[
        pl.BlockSpec((1, NB, OUTW), lambda i: (i, 0, 0)),
        pl.BlockSpec((1, 128), _CONST0),
    ],
    out_shape=[
        jax.ShapeDtypeStruct((GRID, NB, OUTW), jnp.float32),
        jax.ShapeDtypeStruct((1, 128), jnp.float32),
    ],
)


def _tc_compute(x3, g3, Ww, bw, Wa1, ba1, Wa2, ba2):
    return pl.pallas_call(_tc_body, **_TC_GRID_SPEC)(
        x3, g3, Ww, Wa1, Wa2, bw.reshape(SO, 1), ba1.reshape(H, 1),
        ba2.reshape(C, 1), jnp.asarray(_SEL_T), jnp.asarray(_W1))


def kernel(fushed_features, input_data, Ww, bw, Wa1, ba1, Wa2, ba2, adj_idx):
    xpad = jnp.pad(input_data.reshape(B * N, SX), ((0, 0), (0, F - SX)))
    base = (jnp.arange(B, dtype=jnp.int32) * N)[:, None, None]
    idx3 = (adj_idx.astype(jnp.int32) + base).reshape(NW, CH, CW)

    g = _sc_gather(xpad, idx3)

    x3 = xpad.reshape(GRID, NB, F)
    g3 = g.reshape(GRID, MB, F)

    out3, acc = _tc_compute(x3, g3, Ww, bw, Wa1, ba1, Wa2, ba2)

    output_data = out3.reshape(B, N, C, SO)
    cluster_loss = acc[0, 0] / (B * N)
    dist_mean = acc[0, 1] / (B * N * K * K)
    wh_mean = acc[0, 2] / (B * N * SO)
    return output_data, cluster_loss, dist_mean, wh_mean
